# baseline (speedup 1.0000x reference)
import jax
import jax.numpy as jnp
from jax.experimental import pallas as pl

USER = 25000
ITEM = 25000
N = USER + ITEM
LATDIM = 64


def _l2norm(x):
    n = jnp.linalg.norm(x, axis=1, keepdims=True)
    return x / jnp.maximum(n, 1e-12)


def _spmm(idx, vals, X):
    gathered = jnp.take(X, idx[1], axis=0) * vals[:, None]
    return jax.ops.segment_sum(gathered, idx[0], num_segments=N)


def _final_kernel(e0_ref, e1_ref, e2_ref, o_ref):
    e0 = e0_ref[...]
    s = jnp.sum(e0 * e0, axis=1, keepdims=True)
    inv = e0 / jnp.maximum(jnp.sqrt(s), 1e-12)
    o_ref[...] = e0 + e1_ref[...] + e2_ref[...] + 0.2 * inv


def kernel(uEmbeds, iEmbeds, image_embedding, text_embedding, W_img, b_img, W_txt, b_txt,
           modal_weight, adj_vals, image_adj_vals, text_adj_vals,
           adj_idx, image_adj_idx, text_adj_idx):
    image_feats = image_embedding @ W_img + b_img
    text_feats = text_embedding @ W_txt + b_txt
    weight = jax.nn.softmax(modal_weight, axis=0)
    base = jnp.concatenate([uEmbeds, iEmbeds], axis=0)

    embedsImageAdj = _spmm(image_adj_idx, image_adj_vals, base)
    embedsImage = _spmm(adj_idx, adj_vals, jnp.concatenate([uEmbeds, _l2norm(image_feats)], axis=0))
    embedsImage_ = _spmm(adj_idx, adj_vals, jnp.concatenate([embedsImage[:USER], iEmbeds], axis=0))
    embedsImage = embedsImage + embedsImage_

    embedsTextAdj = _spmm(text_adj_idx, text_adj_vals, base)
    embedsText = _spmm(adj_idx, adj_vals, jnp.concatenate([uEmbeds, _l2norm(text_feats)], axis=0))
    embedsText_ = _spmm(adj_idx, adj_vals, jnp.concatenate([embedsText[:USER], iEmbeds], axis=0))
    embedsText = embedsText + embedsText_

    embedsImage = embedsImage + 0.2 * embedsImageAdj
    embedsText = embedsText + 0.2 * embedsTextAdj
    embedsModal = weight[0] * embedsImage + weight[1] * embedsText

    g1 = _spmm(adj_idx, adj_vals, embedsModal)
    g2 = _spmm(adj_idx, adj_vals, g1)

    BM = 1000
    embeds = pl.pallas_call(
        _final_kernel,
        grid=(N // BM,),
        in_specs=[pl.BlockSpec((BM, LATDIM), lambda i: (i, 0))] * 3,
        out_specs=pl.BlockSpec((BM, LATDIM), lambda i: (i, 0)),
        out_shape=jax.ShapeDtypeStruct((N, LATDIM), jnp.float32),
    )(embedsModal, g1, g2)
    return embeds[:USER], embeds[USER:]


# speedup vs baseline: 3.8622x; 3.8622x over previous
import functools

import jax
import jax.numpy as jnp
from jax import lax
from jax.experimental import pallas as pl
from jax.experimental.pallas import tpu as pltpu, tpu_sc as plsc

USER = 25000
ITEM = 25000
N = USER + ITEM
E = 800000
LATDIM = 64
HALF = 32

NTILES = 16
CHUNK = 128
JCHUNKS = 16
GCHUNKS = 25
EDGES_PER_TILE = CHUNK * JCHUNKS * GCHUNKS
E_PAD = EDGES_PER_TILE * NTILES
N_PAD = 50048
ROWS_PER_TILE = N_PAD // NTILES


def _spmm_body(x_hbm, src_hbm, dst_hbm, val_hbm, zeros_hbm, out_hbm,
               acc, sidx, didx, vbuf, rows, sem):
    c = lax.axis_index("c")
    s = lax.axis_index("s")

    pltpu.sync_copy(zeros_hbm.at[pl.ds(s * ROWS_PER_TILE, ROWS_PER_TILE)],
                    acc.at[pl.ds(s * ROWS_PER_TILE, ROWS_PER_TILE)])
    plsc.subcore_barrier()

    row0 = s * (JCHUNKS * GCHUNKS)

    def super_chunk(g, _):
        r0 = row0 + g * JCHUNKS
        pltpu.sync_copy(src_hbm.at[pl.ds(r0, JCHUNKS)], sidx)
        pltpu.sync_copy(dst_hbm.at[pl.ds(r0, JCHUNKS)], didx)
        pltpu.sync_copy(val_hbm.at[pl.ds(r0, JCHUNKS)], vbuf)

        def chunk(j, _):
            pltpu.async_copy(x_hbm.at[c].at[sidx.at[j]], rows, sem).wait()
            for b in range(CHUNK // 16):
                val16 = vbuf[j, pl.ds(b * 16, 16)]
                for t in range(16):
                    r = b * 16 + t
                    v = jnp.full((16,), val16[t], dtype=jnp.float32)
                    rows[r, 0:16] = rows[r, 0:16] * v
                    rows[r, 16:32] = rows[r, 16:32] * v
            pltpu.sync_copy(rows, acc.at[didx.at[j]], add=True)
            return ()

        lax.fori_loop(0, JCHUNKS, chunk, ())
        return ()

    lax.fori_loop(0, GCHUNKS, super_chunk, ())
    plsc.subcore_barrier()
    pltpu.sync_copy(acc.at[pl.ds(s * ROWS_PER_TILE, ROWS_PER_TILE)],
                    out_hbm.at[c].at[pl.ds(s * ROWS_PER_TILE, ROWS_PER_TILE)])


_spmm_call = pl.kernel(
    _spmm_body,
    out_type=jax.ShapeDtypeStruct((2, N_PAD, HALF), jnp.float32),
    mesh=plsc.VectorSubcoreMesh(core_axis_name="c", subcore_axis_name="s"),
    scratch_types=[
        pltpu.VMEM_SHARED((N_PAD, HALF), jnp.float32),
        pltpu.VMEM((JCHUNKS, CHUNK), jnp.int32),
        pltpu.VMEM((JCHUNKS, CHUNK), jnp.int32),
        pltpu.VMEM((JCHUNKS, CHUNK), jnp.float32),
        pltpu.VMEM((CHUNK, HALF), jnp.float32),
        pltpu.SemaphoreType.DMA,
    ],
    compiler_params=pltpu.CompilerParams(use_tc_tiling_on_sc=False),
)


def _prep_edges(idx, vals):
    pad = E_PAD - E
    src = jnp.pad(idx[1], (0, pad)).reshape(-1, CHUNK)
    dst = jnp.pad(idx[0], (0, pad)).reshape(-1, CHUNK)
    val = jnp.pad(vals, (0, pad)).reshape(-1, CHUNK)
    return src, dst, val


def _to_half(x):
    return x.reshape(x.shape[0], 2, HALF).transpose(1, 0, 2)


def _from_half(x2):
    return x2.transpose(1, 0, 2).reshape(x2.shape[1], LATDIM)


def _l2norm(x):
    n = jnp.linalg.norm(x, axis=1, keepdims=True)
    return x / jnp.maximum(n, 1e-12)


def _final_kernel(e0_ref, e1_ref, e2_ref, o_ref):
    e0 = e0_ref[...]
    sq = jnp.sum(e0 * e0, axis=1, keepdims=True)
    o_ref[...] = e0 + e1_ref[...] + e2_ref[...] + 0.2 * (
        e0 / jnp.maximum(jnp.sqrt(sq), 1e-12))


def kernel(uEmbeds, iEmbeds, image_embedding, text_embedding, W_img, b_img, W_txt, b_txt,
           modal_weight, adj_vals, image_adj_vals, text_adj_vals,
           adj_idx, image_adj_idx, text_adj_idx):
    adj = _prep_edges(adj_idx, adj_vals)
    iadj = _prep_edges(image_adj_idx, image_adj_vals)
    tadj = _prep_edges(text_adj_idx, text_adj_vals)

    zeros = jnp.zeros((N_PAD, HALF), jnp.float32)

    def spmm(edges, x2):
        return _spmm_call(x2, *edges, zeros)[:, :N]

    image_feats = image_embedding @ W_img + b_img
    text_feats = text_embedding @ W_txt + b_txt
    weight = jax.nn.softmax(modal_weight, axis=0)

    u2 = _to_half(uEmbeds)
    i2 = _to_half(iEmbeds)
    base2 = jnp.concatenate([u2, i2], axis=1)

    eIAdj = spmm(iadj, base2)
    eI1 = spmm(adj, jnp.concatenate([u2, _to_half(_l2norm(image_feats))], axis=1))
    eI2 = spmm(adj, jnp.concatenate([eI1[:, :USER], i2], axis=1))
    eI = eI1 + eI2

    eTAdj = spmm(tadj, base2)
    eT1 = spmm(adj, jnp.concatenate([u2, _to_half(_l2norm(text_feats))], axis=1))
    eT2 = spmm(adj, jnp.concatenate([eT1[:, :USER], i2], axis=1))
    eT = eT1 + eT2

    eI = eI + 0.2 * eIAdj
    eT = eT + 0.2 * eTAdj
    eModal = weight[0] * eI + weight[1] * eT

    g1 = spmm(adj, eModal)
    g2 = spmm(adj, g1)

    BM = 1000
    embeds = pl.pallas_call(
        _final_kernel,
        grid=(N // BM,),
        in_specs=[pl.BlockSpec((BM, LATDIM), lambda i: (i, 0))] * 3,
        out_specs=pl.BlockSpec((BM, LATDIM), lambda i: (i, 0)),
        out_shape=jax.ShapeDtypeStruct((N, LATDIM), jnp.float32),
    )(_from_half(eModal), _from_half(g1), _from_half(g2))
    return embeds[:USER], embeds[USER:]


# speedup vs baseline: 9.6253x; 2.4922x over previous
import functools

import jax
import jax.numpy as jnp
from jax import lax
from jax.experimental import pallas as pl
from jax.experimental.pallas import tpu as pltpu, tpu_sc as plsc

USER = 25000
ITEM = 25000
N = USER + ITEM
E = 800000
LATDIM = 64
HALF = 32

NTILES = 16
CHUNK = 128
JCHUNKS = 16
GCHUNKS = 25
EDGES_PER_TILE = CHUNK * JCHUNKS * GCHUNKS
E_PAD = EDGES_PER_TILE * NTILES
N_PAD = 50048
ROWS_PER_TILE = N_PAD // NTILES


def _spmm_body(x_hbm, src_hbm, dst_hbm, val_hbm, zeros_hbm, out_hbm,
               acc, sidx, didx, vbuf, rows,
               gsem0, gsem1, gsem2, gsem3, isem_s, isem_d, isem_v, ssem):
    c = lax.axis_index("c")
    s = lax.axis_index("s")
    gsems = (gsem0, gsem1, gsem2, gsem3)

    pltpu.sync_copy(zeros_hbm.at[pl.ds(s * ROWS_PER_TILE, ROWS_PER_TILE)],
                    acc.at[pl.ds(s * ROWS_PER_TILE, ROWS_PER_TILE)])
    plsc.subcore_barrier()

    row0 = s * (JCHUNKS * GCHUNKS)

    def stage_idx(g, p):
        r0 = row0 + g * JCHUNKS
        pltpu.async_copy(src_hbm.at[pl.ds(r0, JCHUNKS)], sidx.at[p], isem_s)
        pltpu.async_copy(dst_hbm.at[pl.ds(r0, JCHUNKS)], didx.at[p], isem_d)
        pltpu.async_copy(val_hbm.at[pl.ds(r0, JCHUNKS)], vbuf.at[p], isem_v)

    def wait_idx(p):
        r0 = row0
        pltpu.make_async_copy(src_hbm.at[pl.ds(r0, JCHUNKS)], sidx.at[p], isem_s).wait()
        pltpu.make_async_copy(dst_hbm.at[pl.ds(r0, JCHUNKS)], didx.at[p], isem_d).wait()
        pltpu.make_async_copy(val_hbm.at[pl.ds(r0, JCHUNKS)], vbuf.at[p], isem_v).wait()

    def fire_gather(p, j, q):
        pltpu.async_copy(x_hbm.at[c].at[sidx.at[p].at[j]], rows.at[q], gsems[q])

    def wait_gather(p, j, q):
        pltpu.make_async_copy(x_hbm.at[c].at[sidx.at[p].at[j]], rows.at[q],
                              gsems[q]).wait()

    def scale_rows(p, j, q):
        def scale16(b, _):
            val16 = vbuf[p, j, pl.ds(b * 16, 16)]
            for t in range(16):
                v = jnp.full((16,), val16[t], dtype=jnp.float32)
                r = b * 16 + t
                rows[q, r, 0:16] = rows[q, r, 0:16] * v
                rows[q, r, 16:32] = rows[q, r, 16:32] * v
            return ()
        lax.fori_loop(0, CHUNK // 16, scale16, ())

    def scatter_add(p, j, q):
        pltpu.sync_copy(rows.at[q], acc.at[didx.at[p].at[j]], add=True)

    stage_idx(0, 0)

    def super_chunk(g, _):
        p = lax.rem(g, 2)
        wait_idx(p)

        @pl.when(g < GCHUNKS - 1)
        def _():
            stage_idx(g + 1, 1 - p)

        for q in range(4):
            fire_gather(p, q, q)

        def group(k, _):
            for q in range(4):
                j = k * 4 + q
                wait_gather(p, j, q)
                scale_rows(p, j, q)
                scatter_add(p, j, q)
                fire_gather(p, j + 4, q)
            return ()

        lax.fori_loop(0, JCHUNKS // 4 - 1, group, ())
        for q in range(4):
            j = JCHUNKS - 4 + q
            wait_gather(p, j, q)
            scale_rows(p, j, q)
            scatter_add(p, j, q)
        return ()

    lax.fori_loop(0, GCHUNKS, super_chunk, ())
    plsc.subcore_barrier()
    pltpu.sync_copy(acc.at[pl.ds(s * ROWS_PER_TILE, ROWS_PER_TILE)],
                    out_hbm.at[c].at[pl.ds(s * ROWS_PER_TILE, ROWS_PER_TILE)])


_spmm_call = pl.kernel(
    _spmm_body,
    out_type=jax.ShapeDtypeStruct((2, N_PAD, HALF), jnp.float32),
    mesh=plsc.VectorSubcoreMesh(core_axis_name="c", subcore_axis_name="s"),
    scratch_types=[
        pltpu.VMEM_SHARED((N_PAD, HALF), jnp.float32),
        pltpu.VMEM((2, JCHUNKS, CHUNK), jnp.int32),
        pltpu.VMEM((2, JCHUNKS, CHUNK), jnp.int32),
        pltpu.VMEM((2, JCHUNKS, CHUNK), jnp.float32),
        pltpu.VMEM((4, CHUNK, HALF), jnp.float32),
    ] + [pltpu.SemaphoreType.DMA] * 8,
    compiler_params=pltpu.CompilerParams(use_tc_tiling_on_sc=False),
)


def _prep_edges(idx, vals):
    pad = E_PAD - E
    fill = jnp.arange(pad, dtype=jnp.int32)
    src = jnp.concatenate([idx[1], fill % N]).reshape(-1, CHUNK)
    dst = jnp.concatenate([idx[0], N + fill % (N_PAD - N)]).reshape(-1, CHUNK)
    val = jnp.pad(vals, (0, pad)).reshape(-1, CHUNK)
    return src, dst, val


def _to_half(x):
    return x.reshape(x.shape[0], 2, HALF).transpose(1, 0, 2)


def _from_half(x2):
    return x2.transpose(1, 0, 2).reshape(x2.shape[1], LATDIM)


def _l2norm(x):
    n = jnp.linalg.norm(x, axis=1, keepdims=True)
    return x / jnp.maximum(n, 1e-12)


def _final_kernel(e0_ref, e1_ref, e2_ref, o_ref):
    e0 = e0_ref[...]
    sq = jnp.sum(e0 * e0, axis=1, keepdims=True)
    o_ref[...] = e0 + e1_ref[...] + e2_ref[...] + 0.2 * (
        e0 / jnp.maximum(jnp.sqrt(sq), 1e-12))


def kernel(uEmbeds, iEmbeds, image_embedding, text_embedding, W_img, b_img, W_txt, b_txt,
           modal_weight, adj_vals, image_adj_vals, text_adj_vals,
           adj_idx, image_adj_idx, text_adj_idx):
    adj = _prep_edges(adj_idx, adj_vals)
    iadj = _prep_edges(image_adj_idx, image_adj_vals)
    tadj = _prep_edges(text_adj_idx, text_adj_vals)

    zeros = jnp.zeros((N_PAD, HALF), jnp.float32)

    def spmm(edges, x2):
        return _spmm_call(x2, *edges, zeros)[:, :N]

    image_feats = image_embedding @ W_img + b_img
    text_feats = text_embedding @ W_txt + b_txt
    weight = jax.nn.softmax(modal_weight, axis=0)

    u2 = _to_half(uEmbeds)
    i2 = _to_half(iEmbeds)
    base2 = jnp.concatenate([u2, i2], axis=1)

    eIAdj = spmm(iadj, base2)
    eI1 = spmm(adj, jnp.concatenate([u2, _to_half(_l2norm(image_feats))], axis=1))
    eI2 = spmm(adj, jnp.concatenate([eI1[:, :USER], i2], axis=1))
    eI = eI1 + eI2

    eTAdj = spmm(tadj, base2)
    eT1 = spmm(adj, jnp.concatenate([u2, _to_half(_l2norm(text_feats))], axis=1))
    eT2 = spmm(adj, jnp.concatenate([eT1[:, :USER], i2], axis=1))
    eT = eT1 + eT2

    eI = eI + 0.2 * eIAdj
    eT = eT + 0.2 * eTAdj
    eModal = weight[0] * eI + weight[1] * eT

    g1 = spmm(adj, eModal)
    g2 = spmm(adj, g1)

    BM = 1000
    embeds = pl.pallas_call(
        _final_kernel,
        grid=(N // BM,),
        in_specs=[pl.BlockSpec((BM, LATDIM), lambda i: (i, 0))] * 3,
        out_specs=pl.BlockSpec((BM, LATDIM), lambda i: (i, 0)),
        out_shape=jax.ShapeDtypeStruct((N, LATDIM), jnp.float32),
    )(_from_half(eModal), _from_half(g1), _from_half(g2))
    return embeds[:USER], embeds[USER:]
